# baseline (speedup 1.0000x reference)
import functools

import jax
import jax.numpy as jnp
from jax import lax
from jax.experimental import pallas as pl
from jax.experimental.pallas import tpu as pltpu
from jax.experimental.pallas import tpu_sc as plsc


def kernel(x, scalar, vector_i, vector_j, vector_k):
    B, L = x.shape
    V, D = scalar.shape
    N = B * L
    T = 4

    info = plsc.get_sparse_core_info()
    NW = info.num_cores * info.num_subcores
    assert N % NW == 0
    n_per_w = N // NW
    C = 256
    assert n_per_w % C == 0
    n_chunks = n_per_w // C

    x_flat = x.reshape(N).astype(jnp.int32)

    mesh = plsc.VectorSubcoreMesh(core_axis_name="c", subcore_axis_name="s")

    @functools.partial(
        pl.kernel,
        mesh=mesh,
        compiler_params=pltpu.CompilerParams(
            needs_layout_passes=False, use_tc_tiling_on_sc=False),
        out_type=jax.ShapeDtypeStruct((N * D * T,), jnp.float32),
        scratch_types=[
            pltpu.VMEM((n_per_w,), jnp.int32),
            pltpu.VMEM((T, C, D), jnp.float32),
            pltpu.VMEM((C * D * T,), jnp.float32),
            pltpu.SemaphoreType.DMA,
        ],
    )
    def sc_kernel(x_hbm, a_hbm, b_hbm, c_hbm, d_hbm, out_hbm,
                  idx_v, rows_v, obuf_v, sem):
        wid = lax.axis_index("s") * info.num_cores + lax.axis_index("c")
        base = wid * n_per_w

        pltpu.sync_copy(x_hbm.at[pl.ds(base, n_per_w)], idx_v)

        lane = lax.broadcasted_iota(jnp.int32, (16,), 0)
        scat = lane * 4

        tabs = (a_hbm, b_hbm, c_hbm, d_hbm)
        row_elems = D * T

        def chunk_body(g, carry):
            idx_slice = idx_v.at[pl.ds(g * C, C)]
            copies = [
                pltpu.async_copy(tabs[t].at[idx_slice], rows_v.at[t], sem)
                for t in range(T)
            ]
            for cp in copies:
                cp.wait()

            def row_body(i, c2):
                for t in range(T):
                    for h in range(D // 16):
                        vals = rows_v[t, i, pl.ds(16 * h, 16)]
                        plsc.store_scatter(
                            obuf_v,
                            [scat + (i * row_elems + 64 * h + t)],
                            vals)
                return c2

            lax.fori_loop(0, C, row_body, 0, unroll=2)

            pltpu.sync_copy(
                obuf_v,
                out_hbm.at[pl.ds((base + g * C) * row_elems, C * row_elems)])
            return carry

        lax.fori_loop(0, n_chunks, chunk_body, 0)

    out = sc_kernel(x_flat, scalar, vector_i, vector_j, vector_k)
    return out.reshape(B, L, D, T)


# speedup vs baseline: 2.8639x; 2.8639x over previous
import functools

import jax
import jax.numpy as jnp
from jax import lax
from jax.experimental import pallas as pl
from jax.experimental.pallas import tpu as pltpu
from jax.experimental.pallas import tpu_sc as plsc


def kernel(x, scalar, vector_i, vector_j, vector_k):
    B, L = x.shape
    V, D = scalar.shape
    N = B * L
    T = 4
    CB = 256
    BB = B // 128

    info = plsc.get_sparse_core_info()
    NW = info.num_cores * info.num_subcores
    assert N % NW == 0 and B % CB == 0
    n_per_w = N // NW
    units_per_w = n_per_w // CB

    x_lt = x.T.reshape(N).astype(jnp.int32)

    mesh = plsc.VectorSubcoreMesh(core_axis_name="c", subcore_axis_name="s")

    @functools.partial(
        pl.kernel,
        mesh=mesh,
        compiler_params=pltpu.CompilerParams(
            needs_layout_passes=False, use_tc_tiling_on_sc=False),
        out_type=jax.ShapeDtypeStruct((L, D, BB, T, 128), jnp.float32),
        scratch_types=[
            pltpu.VMEM((n_per_w,), jnp.int32),
            pltpu.VMEM((T, CB, D), jnp.float32),
            pltpu.VMEM((D, CB // 128, T, 128), jnp.float32),
            pltpu.SemaphoreType.DMA,
        ],
    )
    def sc_kernel(x_hbm, a_hbm, b_hbm, c_hbm, d_hbm, out_hbm,
                  idx_v, rows_v, obuf_v, sem):
        wid = lax.axis_index("s") * info.num_cores + lax.axis_index("c")
        base = wid * n_per_w

        pltpu.sync_copy(x_hbm.at[pl.ds(base, n_per_w)], idx_v)

        lane = lax.broadcasted_iota(jnp.int32, (16,), 0)
        tabs = (a_hbm, b_hbm, c_hbm, d_hbm)
        d_vecs = [lane + 16 * h for h in range(D // 16)]
        t_splats = [jnp.full((16,), t, jnp.int32) for t in range(T)]

        def unit_body(j, carry):
            u = wid * units_per_w + j
            l = u // (B // CB)
            c = u % (B // CB)
            idx_slice = idx_v.at[pl.ds(j * CB, CB)]
            copies = [
                pltpu.async_copy(tabs[t].at[idx_slice], rows_v.at[t], sem)
                for t in range(T)
            ]
            for cp in copies:
                cp.wait()

            def tok_body(i, c2):
                bb = jnp.full((16,), i >> 7, jnp.int32)
                bi = jnp.full((16,), i & 127, jnp.int32)
                for t in range(T):
                    for h in range(D // 16):
                        vals = rows_v[t, i, pl.ds(16 * h, 16)]
                        plsc.store_scatter(
                            obuf_v, [d_vecs[h], bb, t_splats[t], bi], vals)
                return c2

            lax.fori_loop(0, CB, tok_body, 0, unroll=2)

            pltpu.sync_copy(
                obuf_v,
                out_hbm.at[l, :, pl.ds(c * (CB // 128), CB // 128), :, :])
            return carry

        lax.fori_loop(0, units_per_w, unit_body, 0)

    out5 = sc_kernel(x_lt, scalar, vector_i, vector_j, vector_k)
    return out5.transpose(2, 4, 0, 1, 3).reshape(B, L, D, T)


# speedup vs baseline: 2.9750x; 1.0388x over previous
import functools

import jax
import jax.numpy as jnp
from jax import lax
from jax.experimental import pallas as pl
from jax.experimental.pallas import tpu as pltpu
from jax.experimental.pallas import tpu_sc as plsc


def kernel(x, scalar, vector_i, vector_j, vector_k):
    B, L = x.shape
    V, D = scalar.shape
    N = B * L
    T = 4
    CB = 128
    BB = B // 128

    info = plsc.get_sparse_core_info()
    NW = info.num_cores * info.num_subcores
    assert N % NW == 0 and B % CB == 0
    n_per_w = N // NW
    units_per_w = n_per_w // CB
    assert units_per_w % 2 == 0
    cb = B // CB

    x_lt = x.T.reshape(N).astype(jnp.int32)

    mesh = plsc.VectorSubcoreMesh(core_axis_name="c", subcore_axis_name="s")

    @functools.partial(
        pl.kernel,
        mesh=mesh,
        compiler_params=pltpu.CompilerParams(
            needs_layout_passes=False, use_tc_tiling_on_sc=False),
        out_type=jax.ShapeDtypeStruct((L, D, BB, T, 128), jnp.float32),
        scratch_types=[
            pltpu.VMEM((n_per_w,), jnp.int32),
            pltpu.VMEM((2, T, CB, D), jnp.float32),
            pltpu.VMEM((2, D, 1, T, 128), jnp.float32),
            pltpu.SemaphoreType.DMA,
            pltpu.SemaphoreType.DMA,
            pltpu.SemaphoreType.DMA,
            pltpu.SemaphoreType.DMA,
        ],
    )
    def sc_kernel(x_hbm, a_hbm, b_hbm, c_hbm, d_hbm, out_hbm,
                  idx_v, rows2, obuf2, sg0, sg1, so0, so1):
        wid = lax.axis_index("s") * info.num_cores + lax.axis_index("c")
        base = wid * n_per_w
        semg = (sg0, sg1)
        semo = (so0, so1)

        pltpu.sync_copy(x_hbm.at[pl.ds(base, n_per_w)], idx_v)

        lane = lax.broadcasted_iota(jnp.int32, (16,), 0)
        tabs = (a_hbm, b_hbm, c_hbm, d_hbm)
        d_vecs = [lane + 16 * h for h in range(D // 16)]
        t_splats = [jnp.full((16,), t, jnp.int32) for t in range(T)]
        zero16 = jnp.full((16,), 0, jnp.int32)

        def issue_gathers(j, s):
            idx_slice = idx_v.at[pl.ds(j * CB, CB)]
            for t in range(T):
                pltpu.async_copy(tabs[t].at[idx_slice], rows2.at[s, t],
                                 semg[s])

        def wait_gathers(s):
            idx_slice = idx_v.at[pl.ds(0, CB)]
            for t in range(T):
                pltpu.make_async_copy(tabs[t].at[idx_slice], rows2.at[s, t],
                                      semg[s]).wait()

        def out_slice(j):
            u = wid * units_per_w + j
            return out_hbm.at[u // cb, :, pl.ds(u % cb, 1), :, :]

        issue_gathers(0, 0)
        issue_gathers(1, 1)

        def pair_body(p, carry):
            for s in range(2):
                j = 2 * p + s
                wait_gathers(s)

                @pl.when(j >= 2)
                def _():
                    pltpu.make_async_copy(obuf2.at[s], out_slice(j),
                                          semo[s]).wait()

                def tok_body(i, c2):
                    bi = jnp.full((16,), i, jnp.int32)
                    for t in range(T):
                        for h in range(D // 16):
                            vals = rows2[s, t, i, pl.ds(16 * h, 16)]
                            plsc.store_scatter(
                                obuf2.at[s],
                                [d_vecs[h], zero16, t_splats[t], bi], vals)
                    return c2

                lax.fori_loop(0, CB, tok_body, 0, unroll=2)

                pltpu.async_copy(obuf2.at[s], out_slice(j), semo[s])

                @pl.when(j + 2 < units_per_w)
                def _():
                    issue_gathers(j + 2, s)

            return carry

        lax.fori_loop(0, units_per_w // 2, pair_body, 0)

        for s in range(2):
            pltpu.make_async_copy(obuf2.at[s],
                                  out_slice(units_per_w - 2 + s),
                                  semo[s]).wait()

    out5 = sc_kernel(x_lt, scalar, vector_i, vector_j, vector_k)
    return out5.transpose(2, 4, 0, 1, 3).reshape(B, L, D, T)


# speedup vs baseline: 3.1907x; 1.0725x over previous
import functools

import jax
import jax.numpy as jnp
from jax import lax
from jax.experimental import pallas as pl
from jax.experimental.pallas import tpu as pltpu
from jax.experimental.pallas import tpu_sc as plsc


def kernel(x, scalar, vector_i, vector_j, vector_k):
    B, L = x.shape
    V, D = scalar.shape
    N = B * L
    T = 4
    CB = 128
    BB = B // 128

    info = plsc.get_sparse_core_info()
    NW = info.num_cores * info.num_subcores
    assert N % NW == 0 and B % CB == 0
    n_per_w = N // NW
    units_per_w = n_per_w // CB
    assert units_per_w % 2 == 0
    cb = B // CB

    x_lt = x.T.reshape(N).astype(jnp.int32)

    mesh = plsc.VectorSubcoreMesh(core_axis_name="c", subcore_axis_name="s")

    @functools.partial(
        pl.kernel,
        mesh=mesh,
        compiler_params=pltpu.CompilerParams(
            needs_layout_passes=False, use_tc_tiling_on_sc=False),
        out_type=jax.ShapeDtypeStruct((L, D, BB, T, 128), jnp.float32),
        scratch_types=[
            pltpu.VMEM((n_per_w,), jnp.int32),
            pltpu.VMEM((2, T, CB, D), jnp.float32),
            pltpu.VMEM((2, D, 1, T, 128), jnp.float32),
            pltpu.SemaphoreType.DMA,
            pltpu.SemaphoreType.DMA,
            pltpu.SemaphoreType.DMA,
            pltpu.SemaphoreType.DMA,
        ],
    )
    def sc_kernel(x_hbm, a_hbm, b_hbm, c_hbm, d_hbm, out_hbm,
                  idx_v, rows2, obuf2, sg0, sg1, so0, so1):
        wid = lax.axis_index("s") * info.num_cores + lax.axis_index("c")
        base = wid * n_per_w
        semg = (sg0, sg1)
        semo = (so0, so1)

        pltpu.sync_copy(x_hbm.at[pl.ds(base, n_per_w)], idx_v)

        lane = lax.broadcasted_iota(jnp.int32, (16,), 0)
        tabs = (a_hbm, b_hbm, c_hbm, d_hbm)
        d_vecs = [lane + 16 * h for h in range(D // 16)]
        t_splats = [jnp.full((16,), t, jnp.int32) for t in range(T)]
        zero16 = jnp.full((16,), 0, jnp.int32)

        def issue_gathers(j, s):
            idx_slice = idx_v.at[pl.ds(j * CB, CB)]
            for t in range(T):
                pltpu.async_copy(tabs[t].at[idx_slice], rows2.at[s, t],
                                 semg[s])

        def wait_gathers(s):
            idx_slice = idx_v.at[pl.ds(0, CB)]
            for t in range(T):
                pltpu.make_async_copy(tabs[t].at[idx_slice], rows2.at[s, t],
                                      semg[s]).wait()

        def out_slice(j):
            u = wid * units_per_w + j
            return out_hbm.at[u // cb, :, pl.ds(u % cb, 1), :, :]

        issue_gathers(0, 0)
        issue_gathers(1, 1)

        def pair_body(p, carry):
            for s in range(2):
                j = 2 * p + s
                wait_gathers(s)

                @pl.when(j >= 2)
                def _():
                    pltpu.make_async_copy(obuf2.at[s], out_slice(j),
                                          semo[s]).wait()

                @plsc.parallel_loop(0, CB, unroll=4)
                def _(i):
                    bi = jnp.full((16,), i, jnp.int32)
                    for t in range(T):
                        for h in range(D // 16):
                            vals = rows2[s, t, i, pl.ds(16 * h, 16)]
                            plsc.store_scatter(
                                obuf2.at[s],
                                [d_vecs[h], zero16, t_splats[t], bi], vals)

                pltpu.async_copy(obuf2.at[s], out_slice(j), semo[s])

                @pl.when(j + 2 < units_per_w)
                def _():
                    issue_gathers(j + 2, s)

            return carry

        lax.fori_loop(0, units_per_w // 2, pair_body, 0)

        for s in range(2):
            pltpu.make_async_copy(obuf2.at[s],
                                  out_slice(units_per_w - 2 + s),
                                  semo[s]).wait()

    out5 = sc_kernel(x_lt, scalar, vector_i, vector_j, vector_k)
    return out5.transpose(2, 4, 0, 1, 3).reshape(B, L, D, T)
